# baseline (speedup 1.0000x reference)
import functools

import jax
import jax.numpy as jnp
from jax import lax
from jax.experimental import pallas as pl
from jax.experimental.pallas import tpu as pltpu
from jax.experimental.pallas import tpu_sc as plsc

N_NODES = 10000
D_FEAT = 128
N_EDGES = 320000

NC = 2
NS = 16
L = 16

CHUNK = 128
NBUF = 3
CHUNKS_PER_TILE = 81
EDGES_PER_TILE = CHUNK * CHUNKS_PER_TILE
E_PAD = NC * NS * EDGES_PER_TILE
ROWS_PER_TILE = 624
TAIL_ROWS = N_NODES - NS * ROWS_PER_TILE

_mesh = plsc.VectorSubcoreMesh(core_axis_name="c", subcore_axis_name="s")


@functools.partial(
    pl.kernel,
    out_type=jax.ShapeDtypeStruct((NC, N_NODES, D_FEAT), jnp.float32),
    mesh=_mesh,
    scratch_types=[
        pltpu.VMEM((NBUF, 2, CHUNK), jnp.int32),
        pltpu.VMEM((NBUF, CHUNK), jnp.float32),
        pltpu.VMEM((NBUF, CHUNK, D_FEAT), jnp.float32),
        pltpu.VMEM_SHARED((N_NODES, D_FEAT), jnp.float32),
        pltpu.SemaphoreType.DMA,
        pltpu.SemaphoreType.DMA,
        pltpu.SemaphoreType.DMA,
        pltpu.SemaphoreType.DMA,
        pltpu.SemaphoreType.DMA,
        pltpu.SemaphoreType.DMA,
    ],
)
def _sc_spmm(x_hbm, edges_hbm, w_hbm, out_hbm,
             idx_b, w_b, rows_b, acc,
             semg0, semg1, semg2, semi0, semi1, semi2):
    cid = lax.axis_index("c")
    sid = lax.axis_index("s")
    chunk0 = (cid * NS + sid) * CHUNKS_PER_TILE
    semg = (semg0, semg1, semg2)
    semi = (semi0, semi1, semi2)

    zero = jnp.zeros((L,), jnp.float32)

    def zero_row(e, carry):
        for d in range(D_FEAT // L):
            rows_b[0, e, pl.ds(L * d, L)] = zero
        return carry

    lax.fori_loop(0, CHUNK, zero_row, 0, unroll=2)
    row0 = sid * ROWS_PER_TILE
    for k in range(4):
        pltpu.sync_copy(rows_b.at[0], acc.at[pl.ds(row0 + k * CHUNK, CHUNK)])
    pltpu.sync_copy(rows_b.at[0, pl.ds(0, ROWS_PER_TILE - 4 * CHUNK)],
                    acc.at[pl.ds(row0 + 4 * CHUNK, ROWS_PER_TILE - 4 * CHUNK)])

    @pl.when(sid == NS - 1)
    def _zero_tail():
        pltpu.sync_copy(rows_b.at[0, pl.ds(0, TAIL_ROWS)],
                        acc.at[pl.ds(NS * ROWS_PER_TILE, TAIL_ROWS)])

    for m in range(NBUF):
        pltpu.async_copy(edges_hbm.at[chunk0 + m], idx_b.at[m], semi[m])
        pltpu.async_copy(w_hbm.at[chunk0 + m], w_b.at[m], semi[m])
    for m in range(2):
        pltpu.make_async_copy(edges_hbm.at[chunk0 + m],
                              idx_b.at[m], semi[m]).wait()
        pltpu.make_async_copy(w_hbm.at[chunk0 + m],
                              w_b.at[m], semi[m]).wait()
        pltpu.async_copy(x_hbm.at[idx_b.at[m, 0]], rows_b.at[m], semg[m])

    plsc.subcore_barrier()

    def chunk_tri(i, carry):
        for m in range(NBUF):
            g = NBUF * i + m
            m2 = (m + 2) % NBUF

            pltpu.make_async_copy(x_hbm.at[idx_b.at[m, 0]],
                                  rows_b.at[m], semg[m]).wait()

            def q_body(q, carry2, m=m):
                w16 = w_b[m, pl.ds(L * q, L)]
                for j in range(L):
                    e = L * q + j
                    ws = jnp.full((L,), w16[j], jnp.float32)
                    for d in range(D_FEAT // L):
                        sl = pl.ds(L * d, L)
                        rows_b[m, e, sl] = rows_b[m, e, sl] * ws
                return carry2

            lax.fori_loop(0, CHUNK // L, q_body, 0)

            pltpu.sync_copy(rows_b.at[m], acc.at[idx_b.at[m, 1]], add=True)

            @pl.when(g + NBUF < CHUNKS_PER_TILE)
            def _prefetch_idx():
                pltpu.async_copy(edges_hbm.at[chunk0 + g + NBUF],
                                 idx_b.at[m], semi[m])
                pltpu.async_copy(w_hbm.at[chunk0 + g + NBUF],
                                 w_b.at[m], semi[m])

            @pl.when(g + 2 < CHUNKS_PER_TILE)
            def _launch_next():
                pltpu.make_async_copy(edges_hbm.at[chunk0 + g + 2],
                                      idx_b.at[m2], semi[m2]).wait()
                pltpu.make_async_copy(w_hbm.at[chunk0 + g + 2],
                                      w_b.at[m2], semi[m2]).wait()
                pltpu.async_copy(x_hbm.at[idx_b.at[m2, 0]],
                                 rows_b.at[m2], semg[m2])
        return carry

    lax.fori_loop(0, CHUNKS_PER_TILE // NBUF, chunk_tri, 0)

    plsc.subcore_barrier()
    pltpu.sync_copy(acc.at[pl.ds(row0, ROWS_PER_TILE)],
                    out_hbm.at[cid, pl.ds(row0, ROWS_PER_TILE)])

    @pl.when(sid == NS - 1)
    def _write_tail():
        pltpu.sync_copy(acc.at[pl.ds(NS * ROWS_PER_TILE, TAIL_ROWS)],
                        out_hbm.at[cid, pl.ds(NS * ROWS_PER_TILE, TAIL_ROWS)])


def _add_body(a_ref, b_ref, o_ref):
    o_ref[...] = a_ref[...] + b_ref[...]


def _combine(parts):
    blk = 1000
    return pl.pallas_call(
        _add_body,
        grid=(N_NODES // blk,),
        in_specs=[
            pl.BlockSpec((blk, D_FEAT), lambda i: (i, 0)),
            pl.BlockSpec((blk, D_FEAT), lambda i: (i, 0)),
        ],
        out_specs=pl.BlockSpec((blk, D_FEAT), lambda i: (i, 0)),
        out_shape=jax.ShapeDtypeStruct((N_NODES, D_FEAT), jnp.float32),
    )(parts[0], parts[1])


@jax.jit
def kernel(x, edge_index, edge_weight):
    src = edge_index[1].astype(jnp.int32)
    dst = edge_index[0].astype(jnp.int32)
    pad = E_PAD - N_EDGES
    src = jnp.pad(src, (0, pad)).reshape(-1, CHUNK)
    dst = jnp.pad(dst, (0, pad)).reshape(-1, CHUNK)
    w = jnp.pad(edge_weight, (0, pad)).reshape(-1, CHUNK)
    edges = jnp.stack([src, dst], axis=1)
    parts = _sc_spmm(x, edges, w)
    return _combine(parts)


# speedup vs baseline: 1.3816x; 1.3816x over previous
import functools

import jax
import jax.numpy as jnp
from jax import lax
from jax.experimental import pallas as pl
from jax.experimental.pallas import tpu as pltpu
from jax.experimental.pallas import tpu_sc as plsc

N_NODES = 10000
D_FEAT = 128
N_EDGES = 320000

NC = 2
NS = 16
L = 16

CHUNK = 128
CHUNKS_PER_TILE = 80
EDGES_PER_TILE = CHUNK * CHUNKS_PER_TILE
E_PAD = NC * NS * EDGES_PER_TILE
ROWS_PER_TILE = 624
TAIL_ROWS = N_NODES - NS * ROWS_PER_TILE

_mesh = plsc.VectorSubcoreMesh(core_axis_name="c", subcore_axis_name="s")


@functools.partial(
    pl.kernel,
    out_type=jax.ShapeDtypeStruct((NC, N_NODES, D_FEAT), jnp.float32),
    mesh=_mesh,
    scratch_types=[
        pltpu.VMEM((2, 2, CHUNK), jnp.int32),
        pltpu.VMEM((2, CHUNK), jnp.float32),
        pltpu.VMEM((2, CHUNK, D_FEAT), jnp.float32),
        pltpu.VMEM_SHARED((N_NODES, D_FEAT), jnp.float32),
        pltpu.SemaphoreType.DMA,
        pltpu.SemaphoreType.DMA,
        pltpu.SemaphoreType.DMA,
        pltpu.SemaphoreType.DMA,
    ],
)
def _sc_spmm(x_hbm, edges_hbm, w_hbm, out_hbm,
             idx_b, w_b, rows_b, acc, semg0, semg1, semi0, semi1):
    cid = lax.axis_index("c")
    sid = lax.axis_index("s")
    chunk0 = (cid * NS + sid) * CHUNKS_PER_TILE
    semg = (semg0, semg1)
    semi = (semi0, semi1)

    zero = jnp.zeros((L,), jnp.float32)

    def zero_row(e, carry):
        for d in range(D_FEAT // L):
            rows_b[0, e, pl.ds(L * d, L)] = zero
        return carry

    lax.fori_loop(0, CHUNK, zero_row, 0, unroll=2)
    row0 = sid * ROWS_PER_TILE
    for k in range(4):
        pltpu.sync_copy(rows_b.at[0], acc.at[pl.ds(row0 + k * CHUNK, CHUNK)])
    pltpu.sync_copy(rows_b.at[0, pl.ds(0, ROWS_PER_TILE - 4 * CHUNK)],
                    acc.at[pl.ds(row0 + 4 * CHUNK, ROWS_PER_TILE - 4 * CHUNK)])

    @pl.when(sid == NS - 1)
    def _zero_tail():
        pltpu.sync_copy(rows_b.at[0, pl.ds(0, TAIL_ROWS)],
                        acc.at[pl.ds(NS * ROWS_PER_TILE, TAIL_ROWS)])

    pltpu.sync_copy(edges_hbm.at[chunk0], idx_b.at[0])
    pltpu.sync_copy(w_hbm.at[chunk0], w_b.at[0])
    pltpu.async_copy(x_hbm.at[idx_b.at[0, 0]], rows_b.at[0], semg[0])
    pltpu.async_copy(edges_hbm.at[chunk0 + 1], idx_b.at[1], semi[1])
    pltpu.async_copy(w_hbm.at[chunk0 + 1], w_b.at[1], semi[1])

    plsc.subcore_barrier()

    def chunk_pair(i, carry):
        for k in range(2):
            g = 2 * i + k
            kn = 1 - k

            @pl.when(g + 1 < CHUNKS_PER_TILE)
            def _launch_next():
                pltpu.make_async_copy(edges_hbm.at[chunk0 + g + 1],
                                      idx_b.at[kn], semi[kn]).wait()
                pltpu.make_async_copy(w_hbm.at[chunk0 + g + 1],
                                      w_b.at[kn], semi[kn]).wait()
                pltpu.async_copy(x_hbm.at[idx_b.at[kn, 0]],
                                 rows_b.at[kn], semg[kn])

            pltpu.make_async_copy(x_hbm.at[idx_b.at[k, 0]],
                                  rows_b.at[k], semg[k]).wait()

            def q_body(q, carry2, k=k):
                w16 = w_b[k, pl.ds(L * q, L)]
                for j in range(L):
                    e = L * q + j
                    ws = jnp.full((L,), w16[j], jnp.float32)
                    for d in range(D_FEAT // L):
                        sl = pl.ds(L * d, L)
                        rows_b[k, e, sl] = rows_b[k, e, sl] * ws
                return carry2

            lax.fori_loop(0, CHUNK // L, q_body, 0)

            pltpu.sync_copy(rows_b.at[k], acc.at[idx_b.at[k, 1]], add=True)

            @pl.when(g + 2 < CHUNKS_PER_TILE)
            def _prefetch_idx():
                pltpu.async_copy(edges_hbm.at[chunk0 + g + 2],
                                 idx_b.at[k], semi[k])
                pltpu.async_copy(w_hbm.at[chunk0 + g + 2],
                                 w_b.at[k], semi[k])
        return carry

    lax.fori_loop(0, CHUNKS_PER_TILE // 2, chunk_pair, 0)

    plsc.subcore_barrier()
    pltpu.sync_copy(acc.at[pl.ds(row0, ROWS_PER_TILE)],
                    out_hbm.at[cid, pl.ds(row0, ROWS_PER_TILE)])

    @pl.when(sid == NS - 1)
    def _write_tail():
        pltpu.sync_copy(acc.at[pl.ds(NS * ROWS_PER_TILE, TAIL_ROWS)],
                        out_hbm.at[cid, pl.ds(NS * ROWS_PER_TILE, TAIL_ROWS)])


def _add_body(a_ref, b_ref, o_ref):
    o_ref[...] = a_ref[...] + b_ref[...]


def _combine(parts):
    blk = 1000
    return pl.pallas_call(
        _add_body,
        grid=(N_NODES // blk,),
        in_specs=[
            pl.BlockSpec((blk, D_FEAT), lambda i: (i, 0)),
            pl.BlockSpec((blk, D_FEAT), lambda i: (i, 0)),
        ],
        out_specs=pl.BlockSpec((blk, D_FEAT), lambda i: (i, 0)),
        out_shape=jax.ShapeDtypeStruct((N_NODES, D_FEAT), jnp.float32),
    )(parts[0], parts[1])


@jax.jit
def kernel(x, edge_index, edge_weight):
    src = edge_index[1].astype(jnp.int32)
    dst = edge_index[0].astype(jnp.int32)
    pad = E_PAD - N_EDGES
    src = jnp.pad(src, (0, pad)).reshape(-1, CHUNK)
    dst = jnp.concatenate(
        [dst, jnp.arange(pad, dtype=jnp.int32) % N_NODES]).reshape(-1, CHUNK)
    w = jnp.pad(edge_weight, (0, pad)).reshape(-1, CHUNK)
    edges = jnp.stack([src, dst], axis=1)
    parts = _sc_spmm(x, edges, w)
    return _combine(parts)


# speedup vs baseline: 1.4954x; 1.0824x over previous
import functools

import jax
import jax.numpy as jnp
from jax import lax
from jax.experimental import pallas as pl
from jax.experimental.pallas import tpu as pltpu
from jax.experimental.pallas import tpu_sc as plsc

N_NODES = 10000
D_FEAT = 128
N_EDGES = 320000

NC = 2
NS = 16
L = 16

CHUNK = 128
C0 = 120
C1 = 40
TOTAL_CHUNKS = NS * (C0 + C1)
E_PAD = CHUNK * TOTAL_CHUNKS
ROWS_PER_TILE = 624
TAIL_ROWS = N_NODES - NS * ROWS_PER_TILE

_mesh = plsc.VectorSubcoreMesh(core_axis_name="c", subcore_axis_name="s")


@functools.partial(
    pl.kernel,
    out_type=jax.ShapeDtypeStruct((NC, N_NODES, D_FEAT), jnp.float32),
    mesh=_mesh,
    scratch_types=[
        pltpu.VMEM((2, 2, CHUNK), jnp.int32),
        pltpu.VMEM((2, CHUNK), jnp.float32),
        pltpu.VMEM((2, CHUNK, D_FEAT), jnp.float32),
        pltpu.VMEM_SHARED((N_NODES, D_FEAT), jnp.float32),
        pltpu.SemaphoreType.DMA,
        pltpu.SemaphoreType.DMA,
        pltpu.SemaphoreType.DMA,
        pltpu.SemaphoreType.DMA,
    ],
)
def _sc_spmm(x_hbm, edges_hbm, w_hbm, out_hbm,
             idx_b, w_b, rows_b, acc, semg0, semg1, semi0, semi1):
    cid = lax.axis_index("c")
    sid = lax.axis_index("s")
    nch = jnp.where(cid == 0, C0, C1)
    chunk0 = jnp.where(cid == 0, sid * C0, NS * C0 + sid * C1)
    semg = (semg0, semg1)
    semi = (semi0, semi1)

    zero = jnp.zeros((L,), jnp.float32)

    def zero_row(e, carry):
        for d in range(D_FEAT // L):
            rows_b[0, e, pl.ds(L * d, L)] = zero
        return carry

    lax.fori_loop(0, CHUNK, zero_row, 0, unroll=2)
    row0 = sid * ROWS_PER_TILE
    for k in range(4):
        pltpu.sync_copy(rows_b.at[0], acc.at[pl.ds(row0 + k * CHUNK, CHUNK)])
    pltpu.sync_copy(rows_b.at[0, pl.ds(0, ROWS_PER_TILE - 4 * CHUNK)],
                    acc.at[pl.ds(row0 + 4 * CHUNK, ROWS_PER_TILE - 4 * CHUNK)])

    @pl.when(sid == NS - 1)
    def _zero_tail():
        pltpu.sync_copy(rows_b.at[0, pl.ds(0, TAIL_ROWS)],
                        acc.at[pl.ds(NS * ROWS_PER_TILE, TAIL_ROWS)])

    pltpu.sync_copy(edges_hbm.at[chunk0], idx_b.at[0])
    pltpu.sync_copy(w_hbm.at[chunk0], w_b.at[0])
    pltpu.async_copy(x_hbm.at[idx_b.at[0, 0]], rows_b.at[0], semg[0])
    pltpu.async_copy(edges_hbm.at[chunk0 + 1], idx_b.at[1], semi[1])
    pltpu.async_copy(w_hbm.at[chunk0 + 1], w_b.at[1], semi[1])

    plsc.subcore_barrier()

    def chunk_pair(i, carry):
        for k in range(2):
            g = 2 * i + k
            kn = 1 - k

            @pl.when(g + 1 < nch)
            def _launch_next():
                pltpu.make_async_copy(edges_hbm.at[chunk0 + g + 1],
                                      idx_b.at[kn], semi[kn]).wait()
                pltpu.make_async_copy(w_hbm.at[chunk0 + g + 1],
                                      w_b.at[kn], semi[kn]).wait()
                pltpu.async_copy(x_hbm.at[idx_b.at[kn, 0]],
                                 rows_b.at[kn], semg[kn])

            @pl.when(g < nch)
            def _wait_gather():
                pltpu.make_async_copy(x_hbm.at[idx_b.at[k, 0]],
                                      rows_b.at[k], semg[k]).wait()

            def q_body(q, carry2, k=k):
                w16 = w_b[k, pl.ds(L * q, L)]
                for j in range(L):
                    e = L * q + j
                    ws = jnp.full((L,), w16[j], jnp.float32)
                    for d in range(D_FEAT // L):
                        sl = pl.ds(L * d, L)
                        rows_b[k, e, sl] = rows_b[k, e, sl] * ws
                return carry2

            @pl.when(g < nch)
            def _compute_scatter():
                lax.fori_loop(0, CHUNK // L, q_body, 0)
                pltpu.sync_copy(rows_b.at[k], acc.at[idx_b.at[k, 1]],
                                add=True)

            @pl.when(g + 2 < nch)
            def _prefetch_idx():
                pltpu.async_copy(edges_hbm.at[chunk0 + g + 2],
                                 idx_b.at[k], semi[k])
                pltpu.async_copy(w_hbm.at[chunk0 + g + 2],
                                 w_b.at[k], semi[k])
        return carry

    lax.fori_loop(0, C0 // 2, chunk_pair, 0)

    plsc.subcore_barrier()
    pltpu.sync_copy(acc.at[pl.ds(row0, ROWS_PER_TILE)],
                    out_hbm.at[cid, pl.ds(row0, ROWS_PER_TILE)])

    @pl.when(sid == NS - 1)
    def _write_tail():
        pltpu.sync_copy(acc.at[pl.ds(NS * ROWS_PER_TILE, TAIL_ROWS)],
                        out_hbm.at[cid, pl.ds(NS * ROWS_PER_TILE, TAIL_ROWS)])


def _add_body(a_ref, b_ref, o_ref):
    o_ref[...] = a_ref[...] + b_ref[...]


def _combine(parts):
    blk = 1000
    return pl.pallas_call(
        _add_body,
        grid=(N_NODES // blk,),
        in_specs=[
            pl.BlockSpec((blk, D_FEAT), lambda i: (i, 0)),
            pl.BlockSpec((blk, D_FEAT), lambda i: (i, 0)),
        ],
        out_specs=pl.BlockSpec((blk, D_FEAT), lambda i: (i, 0)),
        out_shape=jax.ShapeDtypeStruct((N_NODES, D_FEAT), jnp.float32),
    )(parts[0], parts[1])


@jax.jit
def kernel(x, edge_index, edge_weight):
    src = edge_index[1].astype(jnp.int32)
    dst = edge_index[0].astype(jnp.int32)
    pad = E_PAD - N_EDGES
    src = jnp.pad(src, (0, pad)).reshape(-1, CHUNK)
    dst = jnp.concatenate(
        [dst, jnp.arange(pad, dtype=jnp.int32) % N_NODES]).reshape(-1, CHUNK)
    w = jnp.pad(edge_weight, (0, pad)).reshape(-1, CHUNK)
    edges = jnp.stack([src, dst], axis=1)
    parts = _sc_spmm(x, edges, w)
    return _combine(parts)
